# Initial kernel scaffold; baseline (speedup 1.0000x reference)
import jax
import jax.numpy as jnp
from jax.experimental import pallas as pl


def kernel(x, edge_index, W0_1, W1_1, W2_1, W3_1, W4_1, W0_2, W1_2, W2_2, W3_2, W4_2, W_res):
    raise NotImplementedError("write your pallas kernel here")


# speedup vs baseline: 17.5929x; 17.5929x over previous
import jax
import jax.numpy as jnp
from jax import lax
from jax.experimental import pallas as pl
from jax.experimental.pallas import tpu as pltpu
from jax.experimental.pallas import tpu_sc as plsc

_N = 10000
_E = 160000
_F = 128
_NC = 2
_NS = 16
_NW = _NC * _NS
_EPW = _E // _NW
_KCH = 40
_NCHUNK = _EPW // _KCH
_NP = 10240
_RPT = _NP // _NS


def _scat_body(y_hbm, zeros_hbm, src_hbm, dst_hbm, out_hbm,
               srcv, dstv, rows, acc, sem):
    c = lax.axis_index("c")
    s = lax.axis_index("s")
    w = c * _NS + s
    pltpu.sync_copy(zeros_hbm.at[pl.ds(s * _RPT, _RPT)],
                    acc.at[pl.ds(s * _RPT, _RPT)])
    pltpu.sync_copy(src_hbm.at[w], srcv)
    pltpu.sync_copy(dst_hbm.at[w], dstv)
    plsc.subcore_barrier()

    def chunk(j, carry):
        pltpu.async_copy(y_hbm.at[srcv.at[j]], rows, sem).wait()
        pltpu.sync_copy(rows, acc.at[dstv.at[j]], add=True)
        return carry

    lax.fori_loop(0, _NCHUNK, chunk, 0)
    plsc.subcore_barrier()
    pltpu.sync_copy(acc.at[pl.ds(s * _RPT, _RPT)],
                    out_hbm.at[c, pl.ds(s * _RPT, _RPT)])


_scat = pl.kernel(
    _scat_body,
    out_type=jax.ShapeDtypeStruct((_NC, _NP, _F), jnp.float32),
    mesh=plsc.VectorSubcoreMesh(core_axis_name="c", subcore_axis_name="s",
                                num_cores=_NC, num_subcores=_NS),
    scratch_types=[
        pltpu.VMEM((_NCHUNK, _KCH), jnp.int32),
        pltpu.VMEM((_NCHUNK, _KCH), jnp.int32),
        pltpu.VMEM((_KCH, _F), jnp.float32),
        pltpu.VMEM_SHARED((_NP, _F), jnp.float32),
        pltpu.SemaphoreType.DMA,
    ],
)


def _mm_body(x_ref, w_ref, o_ref):
    o_ref[...] = jnp.dot(x_ref[...], w_ref[...],
                         preferred_element_type=jnp.float32)


def _mm(x, w, bm=400):
    m, k = x.shape
    _, f = w.shape
    return pl.pallas_call(
        _mm_body,
        grid=(m // bm,),
        in_specs=[
            pl.BlockSpec((bm, k), lambda i: (i, 0)),
            pl.BlockSpec((k, f), lambda i: (0, 0)),
        ],
        out_specs=pl.BlockSpec((bm, f), lambda i: (i, 0)),
        out_shape=jax.ShapeDtypeStruct((m, f), jnp.float32),
    )(x, w)


def kernel(x, edge_index, W0_1, W1_1, W2_1, W3_1, W4_1,
           W0_2, W1_2, W2_2, W3_2, W4_2, W_res):
    src = edge_index[0].astype(jnp.int32).reshape(_NW, _NCHUNK, _KCH)
    dst = edge_index[1].astype(jnp.int32).reshape(_NW, _NCHUNK, _KCH)
    zeros = jnp.zeros((_NP, _F), jnp.float32)

    parts = _scat(jnp.ones((_N, _F), jnp.float32), zeros, src, dst)
    deg = parts[0, :_N, 0] + parts[1, :_N, 0] + 1.0
    u = lax.rsqrt(deg)[:, None]

    def prop(z):
        y = u * z
        p = _scat(y, zeros, src, dst)
        return u * (p[0, :_N] + p[1, :_N] + y)

    def P(z):
        return 0.5 * (z + prop(z))

    def layer(h, wc):
        g = _mm(h, wc)
        q0 = prop(g)
        t1 = 0.5 * (g + q0)
        t2 = P(t1)
        t4 = P(P(t2))
        t8 = P(P(P(P(t4))))
        out = jnp.concatenate([
            q0[:, 0:40], (g - t1)[:, 40:60], (t1 - t2)[:, 60:80],
            (t2 - t4)[:, 80:100], (t4 - t8)[:, 100:120],
            jnp.zeros((_N, 8), jnp.float32)], axis=1)
        return jnp.abs(out)

    wc1 = jnp.concatenate(
        [W0_1, W1_1, W2_1, W3_1, W4_1, jnp.zeros((1639, 8), jnp.float32)],
        axis=1)
    h = layer(x, wc1)

    wc2 = jnp.concatenate(
        [W0_2, W1_2, W2_2, W3_2, W4_2, jnp.zeros((120, 8), jnp.float32)],
        axis=1)
    wc2 = jnp.concatenate([wc2, jnp.zeros((8, _F), jnp.float32)], axis=0)
    h = layer(h, wc2)

    wr = jnp.zeros((_F, _F), jnp.float32).at[:120, :4].set(W_res)
    s = _mm(h, wr)
    out = s + 0.1 * prop(s)
    return jax.nn.log_softmax(out[:, :4], axis=1)
